# baseline (speedup 1.0000x reference)
import jax
import jax.numpy as jnp
from jax.experimental import pallas as pl
from jax.experimental.pallas import tpu as pltpu

T = 4096
H = 2048
E = 8
O2 = 4096
OC = 4
CH = O2 // OC


def _body(sp_ref, hid_ref, w_ref, out_ref, sh_ref):
    e = pl.program_id(0)
    oc = pl.program_id(1)

    @pl.when((e == 0) & (oc == 0))
    def _init():
        out_ref[...] = jnp.zeros_like(out_ref)
        sh_ref[...] = jax.lax.dot_general(
            sp_ref[...], hid_ref[...], (((1,), (0,)), ((), ())),
            precision=jax.lax.Precision.HIGHEST,
            preferred_element_type=jnp.float32)

    wsum = jnp.sum(w_ref[0], axis=-1)
    s_e = sh_ref[pl.ds(e, 1), :]
    out_ref[...] += jnp.sum(s_e[0] * wsum).reshape(1, 1)


def kernel(hidden_4d, sparsity, gate_up_proj):
    hidden = hidden_4d.reshape(T, H)
    sp = sparsity.reshape(E, T)
    w = gate_up_proj.reshape(E, H, O2)
    out = pl.pallas_call(
        _body,
        grid=(E, OC),
        in_specs=[
            pl.BlockSpec((E, T), lambda e, oc: (0, 0)),
            pl.BlockSpec((T, H), lambda e, oc: (0, 0)),
            pl.BlockSpec((1, H, CH), lambda e, oc: (e, 0, oc)),
        ],
        out_specs=pl.BlockSpec((1, 1), lambda e, oc: (0, 0)),
        out_shape=jax.ShapeDtypeStruct((1, 1), jnp.float32),
        scratch_shapes=[pltpu.VMEM((E, H), jnp.float32)],
    )(sp, hidden, w)
    return out[0, 0]


# speedup vs baseline: 1.1123x; 1.1123x over previous
import jax
import jax.numpy as jnp
from jax.experimental import pallas as pl
from jax.experimental.pallas import tpu as pltpu

T = 4096
H = 2048
E = 8
O2 = 4096
OC = 4
CH = O2 // OC
NSTEPS = E * OC
TCH = T // NSTEPS


def _body(sp_ref, hid_ref, w_ref, out_ref, sh_ref, ws_ref):
    e = pl.program_id(0)
    oc = pl.program_id(1)
    k = e * OC + oc

    @pl.when(k == 0)
    def _init():
        sh_ref[...] = jnp.zeros_like(sh_ref)
        ws_ref[...] = jnp.zeros_like(ws_ref)

    sp_c = sp_ref[:, pl.ds(k * TCH, TCH)]
    hid_c = hid_ref[pl.ds(k * TCH, TCH), :]
    sh_ref[...] += jax.lax.dot_general(
        sp_c, hid_c, (((1,), (0,)), ((), ())),
        precision=jax.lax.Precision.HIGHEST,
        preferred_element_type=jnp.float32)

    wsum = jnp.sum(w_ref[0], axis=-1)
    ws_ref[pl.ds(e, 1), :] += wsum[None, :]

    @pl.when(k == NSTEPS - 1)
    def _fin():
        out_ref[...] = jnp.sum(sh_ref[...] * ws_ref[...]).reshape(1, 1)


def kernel(hidden_4d, sparsity, gate_up_proj):
    hidden = hidden_4d.reshape(T, H)
    sp = sparsity.reshape(E, T)
    w = gate_up_proj.reshape(E, H, O2)
    out = pl.pallas_call(
        _body,
        grid=(E, OC),
        in_specs=[
            pl.BlockSpec((E, T), lambda e, oc: (0, 0)),
            pl.BlockSpec((T, H), lambda e, oc: (0, 0)),
            pl.BlockSpec((1, H, CH), lambda e, oc: (e, 0, oc)),
        ],
        out_specs=pl.BlockSpec((1, 1), lambda e, oc: (0, 0)),
        out_shape=jax.ShapeDtypeStruct((1, 1), jnp.float32),
        scratch_shapes=[pltpu.VMEM((E, H), jnp.float32),
                        pltpu.VMEM((E, H), jnp.float32)],
    )(sp, hidden, w)
    return out[0, 0]
